# baseline (speedup 1.0000x reference)
import functools

import jax
import jax.numpy as jnp
import numpy as np
from jax import lax
from jax.experimental import pallas as pl
from jax.experimental.pallas import tpu as pltpu
from jax.experimental.pallas import tpu_sc as plsc

B = 4
T = 2048
D = 128
NC, NS, L = 2, 16, 16
NW = NC * NS
TW = T // NW
PER_W = B * TW
NQ = PER_W // 4
EMBD_PDROP = 0.1
_KEEP = 1.0 - EMBD_PDROP
_KINV = float(np.float32(1.0) / np.float32(_KEEP))


def _threefry2x32_np(k1, k2, x0, x1):
    rots = [np.uint32(r) for r in (13, 15, 26, 6, 17, 29, 16, 24)]
    rot0, rot1 = rots[:4], rots[4:]
    ks = [np.uint32(k1), np.uint32(k2),
          np.uint32(np.uint32(k1) ^ np.uint32(k2) ^ np.uint32(0x1BD11BDA))]
    x = [x0.astype(np.uint32), x1.astype(np.uint32)]

    def rotl(v, d):
        return (v << d) | (v >> np.uint32(32 - int(d)))

    def rounds(x, rr):
        for r in rr:
            x[0] = x[0] + x[1]
            x[1] = x[0] ^ rotl(x[1], r)
        return x

    with np.errstate(over="ignore"):
        x[0] = x[0] + ks[0]; x[1] = x[1] + ks[1]
        x = rounds(x, rot0); x[0] += ks[1]; x[1] += ks[2] + np.uint32(1)
        x = rounds(x, rot1); x[0] += ks[2]; x[1] += ks[0] + np.uint32(2)
        x = rounds(x, rot0); x[0] += ks[0]; x[1] += ks[1] + np.uint32(3)
        x = rounds(x, rot1); x[0] += ks[1]; x[1] += ks[2] + np.uint32(4)
        x = rounds(x, rot0); x[0] += ks[2]; x[1] += ks[0] + np.uint32(5)
    return x


def _dropout_keep_mask(seed, keep_prob, shape):
    n = int(np.prod(shape))
    i64 = np.arange(n, dtype=np.uint64)
    c1 = (i64 >> np.uint64(32)).astype(np.uint32)
    c2 = (i64 & np.uint64(0xFFFFFFFF)).astype(np.uint32)
    b1, b2 = _threefry2x32_np(np.uint32((seed >> 32) & 0xFFFFFFFF),
                              np.uint32(seed & 0xFFFFFFFF), c1, c2)
    bits = (b1 ^ b2).reshape(shape)
    fb = (bits >> np.uint32(9)) | np.uint32(0x3F800000)
    floats = fb.view(np.float32) - np.float32(1.0)
    return floats < np.float32(keep_prob)


def _packed_mask_words():
    m = (_dropout_keep_mask(42, _KEEP, (B, T, D))
         .reshape(B, NW, TW, D).transpose(1, 0, 2, 3)
         .reshape(NW, PER_W, D // L, L))
    m = m.reshape(NW, NQ, 4, D // L, L)
    p = (np.arange(4)[:, None] * (D // L)
         + np.arange(D // L)[None, :]).astype(np.uint32)
    words = (m.astype(np.uint32)
             << p[None, None, :, :, None]).sum(axis=(2, 3), dtype=np.uint32)
    return np.ascontiguousarray(words.astype(np.int32))


_MASK_WORDS = _packed_mask_words()

_mesh = plsc.VectorSubcoreMesh(
    core_axis_name="c", subcore_axis_name="s", num_cores=NC, num_subcores=NS)


_MROW = NQ * L // 128


@functools.partial(
    pl.kernel,
    out_type=jax.ShapeDtypeStruct((B, T, D), jnp.float32),
    mesh=_mesh,
    scratch_types=[
        pltpu.VMEM((B, TW), jnp.int32),
        pltpu.VMEM((PER_W, D), jnp.float32),
        pltpu.VMEM((TW, D), jnp.float32),
        pltpu.VMEM((NQ, L), jnp.int32),
        pltpu.SemaphoreType.DMA,
        pltpu.SemaphoreType.DMA,
    ],
)
def _embed(idx_hbm, wte_hbm, wpe_hbm, mask_hbm, out_hbm,
           idx_v, rows_v, wpe_v, mask_v, gsem, ssem):
    wid = lax.axis_index("s") * NC + lax.axis_index("c")
    t0 = wid * TW

    with jax.named_scope("stage_idx"):
        for b in range(B):
            pltpu.sync_copy(idx_hbm.at[b, pl.ds(t0, TW)], idx_v.at[b])
    gathers = [
        pltpu.async_copy(wte_hbm.at[idx_v.at[b]],
                         rows_v.at[pl.ds(b * TW, TW)], gsem)
        for b in range(B)
    ]
    with jax.named_scope("stage_wpe_mask"):
        pltpu.sync_copy(wpe_hbm.at[pl.ds(t0, TW)], wpe_v)
        pltpu.sync_copy(mask_hbm.at[wid], mask_v)

    nq_b = NQ // B
    stores = []
    for b in range(B):
        with jax.named_scope(f"gwait{b}"):
            gathers[b].wait()

        def body(q, carry, b=b):
            bits = mask_v[b * nq_b + q, :]
            r0 = b * TW + 4 * q
            j0 = 4 * q
            for rr in range(4):
                for c in range(D // L):
                    s = pl.ds(c * L, L)
                    bitf = ((bits >> (rr * (D // L) + c)) & 1).astype(
                        jnp.float32)
                    rows_v[r0 + rr, s] = (
                        (rows_v[r0 + rr, s] + wpe_v[j0 + rr, s])
                        * _KINV) * bitf
            return carry

        with jax.named_scope(f"compute{b}"):
            lax.fori_loop(0, nq_b, body, 0, unroll=False)
        stores.append(
            pltpu.async_copy(rows_v.at[pl.ds(b * TW, TW)],
                             out_hbm.at[b, pl.ds(t0, TW)], ssem))
    with jax.named_scope("store_drain"):
        for st in stores:
            st.wait()


def kernel(idx, wte_table, wpe_table):
    mask_words = jnp.asarray(_MASK_WORDS)
    return _embed(idx.astype(jnp.int32), wte_table, wpe_table, mask_words)


# speedup vs baseline: 1.0514x; 1.0514x over previous
import functools

import jax
import jax.numpy as jnp
import numpy as np
from jax import lax
from jax.experimental import pallas as pl
from jax.experimental.pallas import tpu as pltpu
from jax.experimental.pallas import tpu_sc as plsc

B = 4
T = 2048
D = 128
NC, NS, L = 2, 16, 16
NW = NC * NS
TW = T // NW
PER_W = B * TW
NQ = PER_W // 4
EMBD_PDROP = 0.1
_KEEP = 1.0 - EMBD_PDROP
_KINV = float(np.float32(1.0) / np.float32(_KEEP))


def _threefry2x32_np(k1, k2, x0, x1):
    rots = [np.uint32(r) for r in (13, 15, 26, 6, 17, 29, 16, 24)]
    rot0, rot1 = rots[:4], rots[4:]
    ks = [np.uint32(k1), np.uint32(k2),
          np.uint32(np.uint32(k1) ^ np.uint32(k2) ^ np.uint32(0x1BD11BDA))]
    x = [x0.astype(np.uint32), x1.astype(np.uint32)]

    def rotl(v, d):
        return (v << d) | (v >> np.uint32(32 - int(d)))

    def rounds(x, rr):
        for r in rr:
            x[0] = x[0] + x[1]
            x[1] = x[0] ^ rotl(x[1], r)
        return x

    with np.errstate(over="ignore"):
        x[0] = x[0] + ks[0]; x[1] = x[1] + ks[1]
        x = rounds(x, rot0); x[0] += ks[1]; x[1] += ks[2] + np.uint32(1)
        x = rounds(x, rot1); x[0] += ks[2]; x[1] += ks[0] + np.uint32(2)
        x = rounds(x, rot0); x[0] += ks[0]; x[1] += ks[1] + np.uint32(3)
        x = rounds(x, rot1); x[0] += ks[1]; x[1] += ks[2] + np.uint32(4)
        x = rounds(x, rot0); x[0] += ks[2]; x[1] += ks[0] + np.uint32(5)
    return x


def _dropout_keep_mask(seed, keep_prob, shape):
    n = int(np.prod(shape))
    i64 = np.arange(n, dtype=np.uint64)
    c1 = (i64 >> np.uint64(32)).astype(np.uint32)
    c2 = (i64 & np.uint64(0xFFFFFFFF)).astype(np.uint32)
    b1, b2 = _threefry2x32_np(np.uint32((seed >> 32) & 0xFFFFFFFF),
                              np.uint32(seed & 0xFFFFFFFF), c1, c2)
    bits = (b1 ^ b2).reshape(shape)
    fb = (bits >> np.uint32(9)) | np.uint32(0x3F800000)
    floats = fb.view(np.float32) - np.float32(1.0)
    return floats < np.float32(keep_prob)


def _packed_mask_words():
    m = (_dropout_keep_mask(42, _KEEP, (B, T, D))
         .reshape(B, NW, TW, D).transpose(1, 0, 2, 3)
         .reshape(NW, PER_W, D // L, L))
    m = m.reshape(NW, NQ, 4, D // L, L)
    p = (np.arange(4)[:, None] * (D // L)
         + np.arange(D // L)[None, :]).astype(np.uint32)
    words = (m.astype(np.uint32)
             << p[None, None, :, :, None]).sum(axis=(2, 3), dtype=np.uint32)
    return np.ascontiguousarray(words.astype(np.int32))


_MASK_WORDS = _packed_mask_words()

_mesh = plsc.VectorSubcoreMesh(
    core_axis_name="c", subcore_axis_name="s", num_cores=NC, num_subcores=NS)


_MROW = NQ * L // 128


@functools.partial(
    pl.kernel,
    out_type=jax.ShapeDtypeStruct((B, T, D), jnp.float32),
    mesh=_mesh,
    scratch_types=[
        pltpu.VMEM((B, TW), jnp.int32),
        pltpu.VMEM((PER_W, D), jnp.float32),
        pltpu.VMEM((TW, D), jnp.float32),
        pltpu.VMEM((NQ, L), jnp.int32),
        pltpu.SemaphoreType.DMA,
        pltpu.SemaphoreType.DMA,
        [pltpu.SemaphoreType.DMA] * B,
        pltpu.SemaphoreType.DMA,
    ],
)
def _embed(idx_hbm, wte_hbm, wpe_hbm, mask_hbm, out_hbm,
           idx_v, rows_v, wpe_v, mask_v, isem, msem, gsems, ssem):
    wid = lax.axis_index("s") * NC + lax.axis_index("c")
    t0 = wid * TW

    with jax.named_scope("stage_idx"):
        idx_cps = [
            pltpu.async_copy(idx_hbm.at[b, pl.ds(t0, TW)], idx_v.at[b], isem)
            for b in range(B)
        ]
        mask_cp = pltpu.async_copy(mask_hbm.at[wid], mask_v, msem)
        wpe_cp = pltpu.async_copy(wpe_hbm.at[pl.ds(t0, TW)], wpe_v, msem)
        for cp in idx_cps:
            cp.wait()
    gathers = [
        pltpu.async_copy(wte_hbm.at[idx_v.at[b]],
                         rows_v.at[pl.ds(b * TW, TW)], gsems[b])
        for b in range(B)
    ]
    with jax.named_scope("stage_wpe_mask"):
        mask_cp.wait()
        wpe_cp.wait()

    nq_b = NQ // B
    stores = []
    for b in range(B):
        with jax.named_scope(f"gwait{b}"):
            gathers[b].wait()

        def body(q, carry, b=b):
            bits = mask_v[b * nq_b + q, :]
            r0 = b * TW + 4 * q
            j0 = 4 * q
            for rr in range(4):
                for c in range(D // L):
                    s = pl.ds(c * L, L)
                    bitf = ((bits >> (rr * (D // L) + c)) & 1).astype(
                        jnp.float32)
                    rows_v[r0 + rr, s] = (
                        (rows_v[r0 + rr, s] + wpe_v[j0 + rr, s])
                        * _KINV) * bitf
            return carry

        with jax.named_scope(f"compute{b}"):
            lax.fori_loop(0, nq_b, body, 0, unroll=False)
        stores.append(
            pltpu.async_copy(rows_v.at[pl.ds(b * TW, TW)],
                             out_hbm.at[b, pl.ds(t0, TW)], ssem))
    with jax.named_scope("store_drain"):
        for st in stores:
            st.wait()


def kernel(idx, wte_table, wpe_table):
    mask_words = jnp.asarray(_MASK_WORDS)
    return _embed(idx.astype(jnp.int32), wte_table, wpe_table, mask_words)
